# baseline (speedup 1.0000x reference)
import functools

import jax
import jax.numpy as jnp
from jax import lax
from jax.experimental import pallas as pl
from jax.experimental.pallas import tpu as pltpu
from jax.experimental.pallas import tpu_sc as plsc


def kernel(hist, idx, logs):
    B = hist.shape[1]
    V = logs.shape[0] - 1

    info = plsc.get_sparse_core_info()
    NC, NS = info.num_cores, info.num_subcores
    NW = NC * NS
    b_per_w = B // NW

    mesh = plsc.VectorSubcoreMesh(core_axis_name="c", subcore_axis_name="s")

    @functools.partial(
        pl.kernel,
        mesh=mesh,
        out_type=jax.ShapeDtypeStruct((B, V), jnp.float32),
        scratch_types=[
            pltpu.VMEM((V,), jnp.float32),
            pltpu.SemaphoreType.DMA,
        ],
        compiler_params=pltpu.CompilerParams(use_tc_tiling_on_sc=True),
    )
    def bcast(logs_hbm, out_hbm, row_v, sem):
        wid = lax.axis_index("s") * NC + lax.axis_index("c")
        pltpu.sync_copy(logs_hbm.at[pl.ds(0, V)], row_v)
        base = wid * b_per_w
        copies = [
            pltpu.make_async_copy(row_v, out_hbm.at[base + i], sem)
            for i in range(b_per_w)
        ]
        for c in copies:
            c.start()
        for c in copies:
            c.wait()

    return bcast(logs)


# speedup vs baseline: 1.0011x; 1.0011x over previous
import functools

import jax
import jax.numpy as jnp
from jax import lax
from jax.experimental import pallas as pl
from jax.experimental.pallas import tpu as pltpu
from jax.experimental.pallas import tpu_sc as plsc


def kernel(hist, idx, logs):
    B = hist.shape[1]
    V = logs.shape[0] - 1

    info = plsc.get_sparse_core_info()
    NC, NS = info.num_cores, info.num_subcores
    NW = NC * NS
    b_per_w = B // NW

    mesh = plsc.VectorSubcoreMesh(core_axis_name="c", subcore_axis_name="s")

    @functools.partial(
        pl.kernel,
        mesh=mesh,
        scratch_types=[
            pltpu.VMEM((V,), jnp.float32),
            pltpu.SemaphoreType.DMA,
        ],
    )
    def bcast(logs_hbm, out_hbm, row_v, sem):
        wid = lax.axis_index("s") * NC + lax.axis_index("c")
        pltpu.sync_copy(logs_hbm.at[pl.ds(0, V)], row_v)
        base = wid * b_per_w
        copies = [
            pltpu.make_async_copy(row_v, out_hbm.at[base + i], sem)
            for i in range(b_per_w)
        ]
        for c in copies:
            c.start()
        for c in copies:
            c.wait()

    out_ref = jax.new_ref(jax.lax.empty((B, V), jnp.float32))
    bcast(logs, out_ref)
    return out_ref[...]
